# Initial kernel scaffold; baseline (speedup 1.0000x reference)
import jax
import jax.numpy as jnp
from jax.experimental import pallas as pl


def kernel(x, edge_index, W0, b0, W1, b1):
    raise NotImplementedError("write your pallas kernel here")


# speedup vs baseline: 3.2405x; 3.2405x over previous
import functools

import jax
import jax.numpy as jnp
from jax import lax
from jax.experimental import pallas as pl
from jax.experimental.pallas import tpu as pltpu
from jax.experimental.pallas import tpu_sc as plsc

N = 10000
E = 160000
D = 256
HW = 144
NP = 10000
RB = 1000
NRB = N // RB
CHUNK = 80
NS = 16
EPS = E // NS
NCH = EPS // CHUNK
ZR = 125


def _pad16(h, rows):
    lane = lax.broadcasted_iota(jnp.int32, (rows, 16), 1)
    return jnp.where((lane == 0) & (h == 1), 1.0, 0.0).astype(jnp.float32)


def _mm_pad_body(x_ref, w_ref, o_ref):
    y = jnp.dot(x_ref[...], w_ref[...], preferred_element_type=jnp.float32)
    o_ref[...] = jnp.concatenate([y, _pad16(pl.program_id(1), y.shape[0])], axis=1)


def _mm_pad(x, w):
    return pl.pallas_call(
        _mm_pad_body,
        grid=(NRB, 2),
        in_specs=[
            pl.BlockSpec((RB, D), lambda i, h: (i, 0)),
            pl.BlockSpec((D, 128), lambda i, h: (0, h)),
        ],
        out_specs=pl.BlockSpec((RB, HW), lambda i, h: (h * NRB + i, 0)),
        out_shape=jax.ShapeDtypeStruct((2 * N, HW), jnp.float32),
    )(x, w)


def _agg_to_hidden(acca_ref, accb_ref, b_ref):
    a = acca_ref[...]
    bb = accb_ref[...]
    deg = jnp.maximum(bb[:, 128:129], 1.0)
    agg = jnp.concatenate([a[:, :128], bb[:, :128]], axis=1) / deg
    return jnp.maximum(agg + b_ref[...], 0.0)


def _layer_mm_body(acca_ref, accb_ref, b_ref, w_ref, o_ref):
    hid = _agg_to_hidden(acca_ref, accb_ref, b_ref)
    y = jnp.dot(hid, w_ref[...], preferred_element_type=jnp.float32)
    o_ref[...] = jnp.concatenate([y, _pad16(pl.program_id(1), y.shape[0])], axis=1)


def _layer_mm(acc, b, w):
    return pl.pallas_call(
        _layer_mm_body,
        grid=(NRB, 2),
        in_specs=[
            pl.BlockSpec((RB, HW), lambda i, h: (i, 0)),
            pl.BlockSpec((RB, HW), lambda i, h: (NP // RB + i, 0)),
            pl.BlockSpec((1, D), lambda i, h: (0, 0)),
            pl.BlockSpec((D, 128), lambda i, h: (0, h)),
        ],
        out_specs=pl.BlockSpec((RB, HW), lambda i, h: (h * NRB + i, 0)),
        out_shape=jax.ShapeDtypeStruct((2 * N, HW), jnp.float32),
    )(acc, acc, b.reshape(1, D), w)


def _final_body(acca_ref, accb_ref, b_ref, o_ref):
    hid = _agg_to_hidden(acca_ref, accb_ref, b_ref)
    nrm = jnp.sqrt(jnp.sum(hid * hid, axis=1, keepdims=True))
    o_ref[...] = hid / jnp.maximum(nrm, 1e-12)


def _final(acc, b):
    return pl.pallas_call(
        _final_body,
        grid=(NRB,),
        in_specs=[
            pl.BlockSpec((RB, HW), lambda i: (i, 0)),
            pl.BlockSpec((RB, HW), lambda i: (NP // RB + i, 0)),
            pl.BlockSpec((1, D), lambda i: (0, 0)),
        ],
        out_specs=pl.BlockSpec((RB, D), lambda i: (i, 0)),
        out_shape=jax.ShapeDtypeStruct((N, D), jnp.float32),
    )(acc, acc, b.reshape(1, D))


@functools.cache
def _build_sc_agg():
    mesh = plsc.VectorSubcoreMesh(
        core_axis_name="c", subcore_axis_name="s", num_cores=2, num_subcores=NS
    )
    return functools.partial(
        pl.kernel,
        mesh=mesh,
        compiler_params=pltpu.CompilerParams(use_tc_tiling_on_sc=False),
        out_type=jax.ShapeDtypeStruct((2 * NP, HW), jnp.float32),
        scratch_types=[
            pltpu.VMEM((ZR, HW), jnp.float32),
            pltpu.VMEM((CHUNK,), jnp.int32),
            pltpu.VMEM((CHUNK,), jnp.int32),
            pltpu.VMEM((CHUNK, HW), jnp.float32),
            pltpu.VMEM_SHARED((NP, HW), jnp.float32),
            pltpu.SemaphoreType.DMA,
        ],
    )(_sc_agg_body)


def _sc_agg(y, src, dst):
    return _build_sc_agg()(y, src, dst)


def _sc_agg_body(y_hbm, src_hbm, dst_hbm, acc_hbm, zeros_v, sidx_v, didx_v,
                 rows_v, acc_sh, gsem):
    c = lax.axis_index("c")
    s = lax.axis_index("s")

    def _zrow(r, _):
        for k in range(HW // 16):
            zeros_v[r, pl.ds(k * 16, 16)] = jnp.zeros((16,), jnp.float32)
        return 0
    lax.fori_loop(0, ZR, _zrow, 0)
    rows_per_sub = NP // NS
    for j in range(rows_per_sub // ZR):
        pltpu.sync_copy(zeros_v, acc_sh.at[pl.ds(s * rows_per_sub + j * ZR, ZR)])
    plsc.subcore_barrier()

    row_bias = c * N

    def _chunk(ci, _):
        e0 = s * EPS + ci * CHUNK
        pltpu.sync_copy(src_hbm.at[pl.ds(e0, CHUNK)], sidx_v)
        pltpu.sync_copy(dst_hbm.at[pl.ds(e0, CHUNK)], didx_v)
        for k in range(CHUNK // 16):
            sl = pl.ds(k * 16, 16)
            sidx_v[sl] = sidx_v[sl] + row_bias
        pltpu.async_copy(y_hbm.at[sidx_v], rows_v, gsem).wait()
        pltpu.sync_copy(rows_v, acc_sh.at[didx_v], add=True)
        return 0

    lax.fori_loop(0, NCH, _chunk, 0)
    plsc.subcore_barrier()

    pltpu.sync_copy(
        acc_sh.at[pl.ds(s * rows_per_sub, rows_per_sub)],
        acc_hbm.at[pl.ds(c * NP + s * rows_per_sub, rows_per_sub)],
    )


def kernel(x, edge_index, W0, b0, W1, b1):
    src = edge_index[0]
    dst = edge_index[1]
    y0 = _mm_pad(x, W0)
    acc0 = _sc_agg(y0, src, dst)
    y1 = _layer_mm(acc0, b0, W1)
    acc1 = _sc_agg(y1, src, dst)
    return _final(acc1, b1)


# speedup vs baseline: 6.2671x; 1.9340x over previous
import functools

import jax
import jax.numpy as jnp
from jax import lax
from jax.experimental import pallas as pl
from jax.experimental.pallas import tpu as pltpu
from jax.experimental.pallas import tpu_sc as plsc

N = 10000
E = 160000
D = 256
HW = 144
NP = 10000
RB = 1000
NRB = N // RB
CHUNK = 80
NS = 16
EPS = E // NS
NCH = EPS // CHUNK
ZR = 125


def _pad16(h, rows):
    lane = lax.broadcasted_iota(jnp.int32, (rows, 16), 1)
    return jnp.where((lane == 0) & (h == 1), 1.0, 0.0).astype(jnp.float32)


def _mm_pad_body(x_ref, w_ref, o_ref):
    y = jnp.dot(x_ref[...], w_ref[...], preferred_element_type=jnp.float32)
    o_ref[...] = jnp.concatenate([y, _pad16(pl.program_id(1), y.shape[0])], axis=1)


def _mm_pad(x, w):
    return pl.pallas_call(
        _mm_pad_body,
        grid=(NRB, 2),
        in_specs=[
            pl.BlockSpec((RB, D), lambda i, h: (i, 0)),
            pl.BlockSpec((D, 128), lambda i, h: (0, h)),
        ],
        out_specs=pl.BlockSpec((RB, HW), lambda i, h: (h * NRB + i, 0)),
        out_shape=jax.ShapeDtypeStruct((2 * N, HW), jnp.float32),
    )(x, w)


def _agg_to_hidden(acca_ref, accb_ref, b_ref):
    a = acca_ref[...]
    bb = accb_ref[...]
    deg = jnp.maximum(bb[:, 128:129], 1.0)
    agg = jnp.concatenate([a[:, :128], bb[:, :128]], axis=1) / deg
    return jnp.maximum(agg + b_ref[...], 0.0)


def _layer_mm_body(acca_ref, accb_ref, b_ref, w_ref, o_ref):
    hid = _agg_to_hidden(acca_ref, accb_ref, b_ref)
    y = jnp.dot(hid, w_ref[...], preferred_element_type=jnp.float32)
    o_ref[...] = jnp.concatenate([y, _pad16(pl.program_id(1), y.shape[0])], axis=1)


def _layer_mm(acc, b, w):
    return pl.pallas_call(
        _layer_mm_body,
        grid=(NRB, 2),
        in_specs=[
            pl.BlockSpec((RB, HW), lambda i, h: (i, 0)),
            pl.BlockSpec((RB, HW), lambda i, h: (NP // RB + i, 0)),
            pl.BlockSpec((1, D), lambda i, h: (0, 0)),
            pl.BlockSpec((D, 128), lambda i, h: (0, h)),
        ],
        out_specs=pl.BlockSpec((RB, HW), lambda i, h: (h * NRB + i, 0)),
        out_shape=jax.ShapeDtypeStruct((2 * N, HW), jnp.float32),
    )(acc, acc, b.reshape(1, D), w)


def _final_body(acca_ref, accb_ref, b_ref, o_ref):
    hid = _agg_to_hidden(acca_ref, accb_ref, b_ref)
    nrm = jnp.sqrt(jnp.sum(hid * hid, axis=1, keepdims=True))
    o_ref[...] = hid / jnp.maximum(nrm, 1e-12)


def _final(acc, b):
    return pl.pallas_call(
        _final_body,
        grid=(NRB,),
        in_specs=[
            pl.BlockSpec((RB, HW), lambda i: (i, 0)),
            pl.BlockSpec((RB, HW), lambda i: (NP // RB + i, 0)),
            pl.BlockSpec((1, D), lambda i: (0, 0)),
        ],
        out_specs=pl.BlockSpec((RB, D), lambda i: (i, 0)),
        out_shape=jax.ShapeDtypeStruct((N, D), jnp.float32),
    )(acc, acc, b.reshape(1, D))


RK = 3
KI = 8
GA = 2


@functools.cache
def _build_sc_agg():
    mesh = plsc.VectorSubcoreMesh(
        core_axis_name="c", subcore_axis_name="s", num_cores=2, num_subcores=NS
    )
    return functools.partial(
        pl.kernel,
        mesh=mesh,
        compiler_params=pltpu.CompilerParams(use_tc_tiling_on_sc=False),
        out_type=jax.ShapeDtypeStruct((2 * NP, HW), jnp.float32),
        scratch_types=[
            pltpu.VMEM((KI, CHUNK), jnp.int32),
            pltpu.VMEM((KI, CHUNK), jnp.int32),
            pltpu.VMEM((RK, CHUNK, HW), jnp.float32),
            pltpu.VMEM_SHARED((NP, HW), jnp.float32),
            pltpu.SemaphoreType.DMA((KI,)),
            pltpu.SemaphoreType.DMA((RK,)),
        ],
    )(_sc_agg_body)


def _sc_agg(y, src, dst):
    return _build_sc_agg()(y, src.reshape(NS, NCH, CHUNK),
                           dst.reshape(NS, NCH, CHUNK))


def _sc_agg_body(y_hbm, src_hbm, dst_hbm, acc_hbm, sloc, dloc,
                 rows_v, acc_sh, isem, gsem):
    c = lax.axis_index("c")
    s = lax.axis_index("s")
    row_bias = c * N
    rows_per_sub = NP // NS

    def _fire_idx(ci, slot):
        pltpu.async_copy(src_hbm.at[s, ci], sloc.at[slot], isem.at[slot])
        pltpu.async_copy(dst_hbm.at[s, ci], dloc.at[slot], isem.at[slot])

    def _wait_bias_fire_gather(ci, islot, gslot):
        pltpu.make_async_copy(src_hbm.at[s, 0], sloc.at[islot], isem.at[islot]).wait()
        pltpu.make_async_copy(dst_hbm.at[s, 0], dloc.at[islot], isem.at[islot]).wait()
        for k in range(CHUNK // 16):
            sl = pl.ds(k * 16, 16)
            sloc[islot, sl] = sloc[islot, sl] + row_bias
        pltpu.async_copy(y_hbm.at[sloc.at[islot]], rows_v.at[gslot], gsem.at[gslot])

    def _zrow(r, _):
        for k in range(HW // 16):
            rows_v[0, r, pl.ds(k * 16, 16)] = jnp.zeros((16,), jnp.float32)
        return 0
    lax.fori_loop(0, CHUNK, _zrow, 0)
    zbase = s * rows_per_sub
    for j in range(rows_per_sub // CHUNK):
        pltpu.sync_copy(rows_v.at[0], acc_sh.at[pl.ds(zbase + j * CHUNK, CHUNK)])
    zrem = rows_per_sub % CHUNK
    pltpu.sync_copy(
        rows_v.at[0].at[pl.ds(0, zrem)],
        acc_sh.at[pl.ds(zbase + rows_per_sub - zrem, zrem)],
    )
    plsc.subcore_barrier()

    for q in range(KI):
        _fire_idx(q, q)
    for q in range(GA):
        _wait_bias_fire_gather(q, q, q)

    def _visit(ci, _):
        g = lax.rem(ci, RK)
        pltpu.make_async_copy(y_hbm.at[sloc.at[0]], rows_v.at[g], gsem.at[g]).wait()
        pltpu.sync_copy(rows_v.at[g], acc_sh.at[dloc.at[lax.rem(ci, KI)]], add=True)

        @pl.when(ci + KI < NCH)
        def _():
            _fire_idx(ci + KI, lax.rem(ci, KI))

        @pl.when(ci + GA < NCH)
        def _():
            nxt = ci + GA
            _wait_bias_fire_gather(nxt, lax.rem(nxt, KI), lax.rem(nxt, RK))
        return 0

    lax.fori_loop(0, NCH, _visit, 0)
    plsc.subcore_barrier()

    pltpu.sync_copy(
        acc_sh.at[pl.ds(s * rows_per_sub, rows_per_sub)],
        acc_hbm.at[pl.ds(c * NP + s * rows_per_sub, rows_per_sub)],
    )


def kernel(x, edge_index, W0, b0, W1, b1):
    src = edge_index[0]
    dst = edge_index[1]
    y0 = _mm_pad(x, W0)
    acc0 = _sc_agg(y0, src, dst)
    y1 = _layer_mm(acc0, b0, W1)
    acc1 = _sc_agg(y1, src, dst)
    return _final(acc1, b1)


# speedup vs baseline: 9.6142x; 1.5341x over previous
import functools

import jax
import jax.numpy as jnp
from jax import lax
from jax.experimental import pallas as pl
from jax.experimental.pallas import tpu as pltpu
from jax.experimental.pallas import tpu_sc as plsc

N = 10000
E = 160000
D = 256
HW = 128
ND = 10240
RB = 1000
NRB = N // RB
CHUNK = 80
NS = 16
EPS = E // NS
NCH = EPS // CHUNK
RK = 4
KI = 8
GA = 3


def _mm_body(x_ref, w_ref, o_ref):
    o_ref[...] = jnp.dot(x_ref[...], w_ref[...],
                         preferred_element_type=jnp.float32)


def _mm(x, w):
    return pl.pallas_call(
        _mm_body,
        grid=(NRB, 2),
        in_specs=[
            pl.BlockSpec((RB, D), lambda i, h: (i, 0)),
            pl.BlockSpec((D, HW), lambda i, h: (0, h)),
        ],
        out_specs=pl.BlockSpec((RB, HW), lambda i, h: (h * NRB + i, 0)),
        out_shape=jax.ShapeDtypeStruct((2 * N, HW), jnp.float32),
    )(x, w)


def _agg_to_hidden(acca_ref, accb_ref, deg_ref, b_ref):
    agg = jnp.concatenate([acca_ref[...], accb_ref[...]], axis=1)
    agg = agg / jnp.maximum(deg_ref[...], 1.0)
    return jnp.maximum(agg + b_ref[...], 0.0)


def _layer_mm_body(acca_ref, accb_ref, deg_ref, b_ref, w_ref, o_ref):
    hid = _agg_to_hidden(acca_ref, accb_ref, deg_ref, b_ref)
    o_ref[...] = jnp.dot(hid, w_ref[...], preferred_element_type=jnp.float32)


def _layer_mm(acc, deg, b, w):
    return pl.pallas_call(
        _layer_mm_body,
        grid=(NRB, 2),
        in_specs=[
            pl.BlockSpec((RB, HW), lambda i, h: (i, 0)),
            pl.BlockSpec((RB, HW), lambda i, h: (NRB + i, 0)),
            pl.BlockSpec((RB, 1), lambda i, h: (i, 0)),
            pl.BlockSpec((1, D), lambda i, h: (0, 0)),
            pl.BlockSpec((D, HW), lambda i, h: (0, h)),
        ],
        out_specs=pl.BlockSpec((RB, HW), lambda i, h: (h * NRB + i, 0)),
        out_shape=jax.ShapeDtypeStruct((2 * N, HW), jnp.float32),
    )(acc, acc, deg, b.reshape(1, D), w)


def _final_body(acca_ref, accb_ref, deg_ref, b_ref, o_ref):
    hid = _agg_to_hidden(acca_ref, accb_ref, deg_ref, b_ref)
    nrm = jnp.sqrt(jnp.sum(hid * hid, axis=1, keepdims=True))
    o_ref[...] = hid / jnp.maximum(nrm, 1e-12)


def _final(acc, deg, b):
    return pl.pallas_call(
        _final_body,
        grid=(NRB,),
        in_specs=[
            pl.BlockSpec((RB, HW), lambda i: (i, 0)),
            pl.BlockSpec((RB, HW), lambda i: (NRB + i, 0)),
            pl.BlockSpec((RB, 1), lambda i: (i, 0)),
            pl.BlockSpec((1, D), lambda i: (0, 0)),
        ],
        out_specs=pl.BlockSpec((RB, D), lambda i: (i, 0)),
        out_shape=jax.ShapeDtypeStruct((N, D), jnp.float32),
    )(acc, acc, deg, b.reshape(1, D))


@functools.cache
def _build_sc_agg():
    mesh = plsc.VectorSubcoreMesh(
        core_axis_name="c", subcore_axis_name="s", num_cores=2, num_subcores=NS
    )
    return functools.partial(
        pl.kernel,
        mesh=mesh,
        compiler_params=pltpu.CompilerParams(use_tc_tiling_on_sc=False),
        out_type=(
            jax.ShapeDtypeStruct((2 * N, HW), jnp.float32),
            jax.ShapeDtypeStruct((ND,), jnp.float32),
        ),
        scratch_types=[
            pltpu.VMEM((KI, CHUNK), jnp.int32),
            pltpu.VMEM((KI, CHUNK), jnp.int32),
            pltpu.VMEM((RK, CHUNK, HW), jnp.float32),
            pltpu.VMEM((CHUNK,), jnp.float32),
            pltpu.VMEM((ND // NS,), jnp.float32),
            pltpu.VMEM_SHARED((N, HW), jnp.float32),
            pltpu.VMEM_SHARED((ND,), jnp.float32),
            pltpu.SemaphoreType.DMA((KI,)),
            pltpu.SemaphoreType.DMA((RK,)),
        ],
    )(_sc_agg_body)


def _sc_agg(y, src, dst):
    return _build_sc_agg()(y, src, dst)


def _sc_agg_body(y_hbm, src_hbm, dst_hbm, acc_hbm, deg_hbm, sloc, dloc,
                 rows_v, ones_v, zeros_v, acc_sh, deg_sh, isem, gsem):
    c = lax.axis_index("c")
    s = lax.axis_index("s")
    row_bias = c * N
    rows_per_sub = N // NS
    deg_per_sub = ND // NS

    def _fire_idx(ci, slot):
        e0 = s * EPS + ci * CHUNK
        pltpu.async_copy(src_hbm.at[pl.ds(e0, CHUNK)], sloc.at[slot],
                         isem.at[slot])
        pltpu.async_copy(dst_hbm.at[pl.ds(e0, CHUNK)], dloc.at[slot],
                         isem.at[slot])

    def _wait_bias_fire_gather(islot, gslot):
        pltpu.make_async_copy(src_hbm.at[pl.ds(0, CHUNK)], sloc.at[islot],
                              isem.at[islot]).wait()
        pltpu.make_async_copy(dst_hbm.at[pl.ds(0, CHUNK)], dloc.at[islot],
                              isem.at[islot]).wait()
        for k in range(CHUNK // 16):
            sl = pl.ds(k * 16, 16)
            sloc[islot, sl] = sloc[islot, sl] + row_bias
        pltpu.async_copy(y_hbm.at[sloc.at[islot]], rows_v.at[gslot],
                         gsem.at[gslot])

    for k in range(CHUNK // 16):
        ones_v[pl.ds(k * 16, 16)] = jnp.full((16,), 1.0, jnp.float32)

    def _zrow(r, _):
        for k in range(HW // 16):
            rows_v[0, r, pl.ds(k * 16, 16)] = jnp.zeros((16,), jnp.float32)
        return 0
    lax.fori_loop(0, CHUNK, _zrow, 0)

    def _zdeg(r, _):
        zeros_v[pl.ds(r * 16, 16)] = jnp.zeros((16,), jnp.float32)
        return 0
    lax.fori_loop(0, deg_per_sub // 16, _zdeg, 0)

    zbase = s * rows_per_sub
    for j in range(rows_per_sub // CHUNK):
        pltpu.sync_copy(rows_v.at[0], acc_sh.at[pl.ds(zbase + j * CHUNK, CHUNK)])
    zrem = rows_per_sub % CHUNK
    pltpu.sync_copy(
        rows_v.at[0].at[pl.ds(0, zrem)],
        acc_sh.at[pl.ds(zbase + rows_per_sub - zrem, zrem)],
    )

    @pl.when(c == 1)
    def _():
        pltpu.sync_copy(zeros_v, deg_sh.at[pl.ds(s * deg_per_sub, deg_per_sub)])
    plsc.subcore_barrier()

    for q in range(KI):
        _fire_idx(q, q)
    for q in range(GA):
        _wait_bias_fire_gather(q, q)

    def _visit(ci, _):
        g = lax.rem(ci, RK)
        islot = lax.rem(ci, KI)
        pltpu.make_async_copy(y_hbm.at[sloc.at[0]], rows_v.at[g],
                              gsem.at[g]).wait()
        pltpu.sync_copy(rows_v.at[g], acc_sh.at[dloc.at[islot]], add=True)

        @pl.when(c == 1)
        def _():
            pltpu.sync_copy(ones_v, deg_sh.at[dloc.at[islot]], add=True)

        @pl.when(ci + KI < NCH)
        def _():
            _fire_idx(ci + KI, islot)

        @pl.when(ci + GA < NCH)
        def _():
            nxt = ci + GA
            _wait_bias_fire_gather(lax.rem(nxt, KI), lax.rem(nxt, RK))
        return 0

    lax.fori_loop(0, NCH, _visit, 0)
    plsc.subcore_barrier()

    pltpu.sync_copy(
        acc_sh.at[pl.ds(s * rows_per_sub, rows_per_sub)],
        acc_hbm.at[pl.ds(c * N + s * rows_per_sub, rows_per_sub)],
    )

    @pl.when(c == 1)
    def _():
        pltpu.sync_copy(deg_sh.at[pl.ds(s * deg_per_sub, deg_per_sub)],
                        deg_hbm.at[pl.ds(s * deg_per_sub, deg_per_sub)])


def kernel(x, edge_index, W0, b0, W1, b1):
    src = edge_index[0]
    dst = edge_index[1]
    y0 = _mm(x, W0)
    acc0, deg = _sc_agg(y0, src, dst)
    deg_col = deg[:N, None]
    y1 = _layer_mm(acc0, deg_col, b0, W1)
    acc1, _ = _sc_agg(y1, src, dst)
    return _final(acc1, deg_col, b1)


# speedup vs baseline: 10.3470x; 1.0762x over previous
import functools

import jax
import jax.numpy as jnp
from jax import lax
from jax.experimental import pallas as pl
from jax.experimental.pallas import tpu as pltpu
from jax.experimental.pallas import tpu_sc as plsc

N = 10000
E = 160000
D = 256
HW = 128
ND = 10240
RB = 1000
NRB = N // RB
CHUNK = 80
NS = 16
EPS = E // NS
NCH = EPS // CHUNK
RK = 4
KI = 8
GA = 3


def _mm_body(x_ref, w_ref, oa_ref, ob_ref):
    y = jnp.dot(x_ref[...], w_ref[...], preferred_element_type=jnp.float32)
    oa_ref[...] = y[:, :HW]
    ob_ref[...] = y[:, HW:]


def _mm(x, w):
    return pl.pallas_call(
        _mm_body,
        grid=(NRB,),
        in_specs=[
            pl.BlockSpec((RB, D), lambda i: (i, 0)),
            pl.BlockSpec((D, D), lambda i: (0, 0)),
        ],
        out_specs=[
            pl.BlockSpec((RB, HW), lambda i: (i, 0)),
            pl.BlockSpec((RB, HW), lambda i: (i, 0)),
        ],
        out_shape=[
            jax.ShapeDtypeStruct((N, HW), jnp.float32),
            jax.ShapeDtypeStruct((N, HW), jnp.float32),
        ],
    )(x, w)


def _agg_to_hidden(acca_ref, accb_ref, deg_ref, b_ref):
    agg = jnp.concatenate([acca_ref[...], accb_ref[...]], axis=1)
    agg = agg / jnp.maximum(deg_ref[...], 1.0)
    return jnp.maximum(agg + b_ref[...], 0.0)


def _layer_mm_body(acca_ref, accb_ref, deg_ref, b_ref, w_ref, oa_ref, ob_ref):
    hid = _agg_to_hidden(acca_ref, accb_ref, deg_ref, b_ref)
    y = jnp.dot(hid, w_ref[...], preferred_element_type=jnp.float32)
    oa_ref[...] = y[:, :HW]
    ob_ref[...] = y[:, HW:]


def _layer_mm(acca, accb, deg, b, w):
    return pl.pallas_call(
        _layer_mm_body,
        grid=(NRB,),
        in_specs=[
            pl.BlockSpec((RB, HW), lambda i: (i, 0)),
            pl.BlockSpec((RB, HW), lambda i: (i, 0)),
            pl.BlockSpec((RB, 1), lambda i: (i, 0)),
            pl.BlockSpec((1, D), lambda i: (0, 0)),
            pl.BlockSpec((D, D), lambda i: (0, 0)),
        ],
        out_specs=[
            pl.BlockSpec((RB, HW), lambda i: (i, 0)),
            pl.BlockSpec((RB, HW), lambda i: (i, 0)),
        ],
        out_shape=[
            jax.ShapeDtypeStruct((N, HW), jnp.float32),
            jax.ShapeDtypeStruct((N, HW), jnp.float32),
        ],
    )(acca, accb, deg, b.reshape(1, D), w)


def _final_body(acca_ref, accb_ref, deg_ref, b_ref, o_ref):
    hid = _agg_to_hidden(acca_ref, accb_ref, deg_ref, b_ref)
    nrm = jnp.sqrt(jnp.sum(hid * hid, axis=1, keepdims=True))
    o_ref[...] = hid / jnp.maximum(nrm, 1e-12)


def _final(acca, accb, deg, b):
    return pl.pallas_call(
        _final_body,
        grid=(NRB,),
        in_specs=[
            pl.BlockSpec((RB, HW), lambda i: (i, 0)),
            pl.BlockSpec((RB, HW), lambda i: (i, 0)),
            pl.BlockSpec((RB, 1), lambda i: (i, 0)),
            pl.BlockSpec((1, D), lambda i: (0, 0)),
        ],
        out_specs=pl.BlockSpec((RB, D), lambda i: (i, 0)),
        out_shape=jax.ShapeDtypeStruct((N, D), jnp.float32),
    )(acca, accb, deg, b.reshape(1, D))


@functools.cache
def _build_sc_agg():
    mesh = plsc.VectorSubcoreMesh(
        core_axis_name="c", subcore_axis_name="s", num_cores=2, num_subcores=NS
    )
    return functools.partial(
        pl.kernel,
        mesh=mesh,
        compiler_params=pltpu.CompilerParams(use_tc_tiling_on_sc=False),
        out_type=(
            jax.ShapeDtypeStruct((N, HW), jnp.float32),
            jax.ShapeDtypeStruct((N, HW), jnp.float32),
            jax.ShapeDtypeStruct((ND,), jnp.float32),
        ),
        scratch_types=[
            pltpu.VMEM((KI, CHUNK), jnp.int32),
            pltpu.VMEM((KI, CHUNK), jnp.int32),
            pltpu.VMEM((RK, CHUNK, HW), jnp.float32),
            pltpu.VMEM((CHUNK,), jnp.float32),
            pltpu.VMEM((ND // NS,), jnp.float32),
            pltpu.VMEM_SHARED((N, HW), jnp.float32),
            pltpu.VMEM_SHARED((ND,), jnp.float32),
            pltpu.SemaphoreType.DMA((KI,)),
            pltpu.SemaphoreType.DMA((RK,)),
        ],
    )(_sc_agg_body)


def _sc_agg(ya, yb, src, dst):
    return _build_sc_agg()(ya, yb, src, dst)


def _sc_agg_body(ya_hbm, yb_hbm, src_hbm, dst_hbm, acca_hbm, accb_hbm,
                 deg_hbm, sloc, dloc, rows_v, ones_v, zeros_v, acc_sh,
                 deg_sh, isem, gsem):
    c = lax.axis_index("c")
    s = lax.axis_index("s")
    rows_per_sub = N // NS
    deg_per_sub = ND // NS

    def _fire_idx(ci, slot):
        e0 = s * EPS + ci * CHUNK
        pltpu.async_copy(src_hbm.at[pl.ds(e0, CHUNK)], sloc.at[slot],
                         isem.at[slot])
        pltpu.async_copy(dst_hbm.at[pl.ds(e0, CHUNK)], dloc.at[slot],
                         isem.at[slot])

    def _wait_fire_gather(islot, gslot):
        pltpu.make_async_copy(src_hbm.at[pl.ds(0, CHUNK)], sloc.at[islot],
                              isem.at[islot]).wait()
        pltpu.make_async_copy(dst_hbm.at[pl.ds(0, CHUNK)], dloc.at[islot],
                              isem.at[islot]).wait()

        @pl.when(c == 0)
        def _():
            pltpu.async_copy(ya_hbm.at[sloc.at[islot]], rows_v.at[gslot],
                             gsem.at[gslot])

        @pl.when(c == 1)
        def _():
            pltpu.async_copy(yb_hbm.at[sloc.at[islot]], rows_v.at[gslot],
                             gsem.at[gslot])

    for k in range(CHUNK // 16):
        ones_v[pl.ds(k * 16, 16)] = jnp.full((16,), 1.0, jnp.float32)

    def _zrow(r, _):
        for k in range(HW // 16):
            rows_v[0, r, pl.ds(k * 16, 16)] = jnp.zeros((16,), jnp.float32)
        return 0
    lax.fori_loop(0, CHUNK, _zrow, 0)

    def _zdeg(r, _):
        zeros_v[pl.ds(r * 16, 16)] = jnp.zeros((16,), jnp.float32)
        return 0
    lax.fori_loop(0, deg_per_sub // 16, _zdeg, 0)

    zbase = s * rows_per_sub
    for j in range(rows_per_sub // CHUNK):
        pltpu.sync_copy(rows_v.at[0], acc_sh.at[pl.ds(zbase + j * CHUNK, CHUNK)])
    zrem = rows_per_sub % CHUNK
    pltpu.sync_copy(
        rows_v.at[0].at[pl.ds(0, zrem)],
        acc_sh.at[pl.ds(zbase + rows_per_sub - zrem, zrem)],
    )

    @pl.when(c == 1)
    def _():
        pltpu.sync_copy(zeros_v, deg_sh.at[pl.ds(s * deg_per_sub, deg_per_sub)])
    plsc.subcore_barrier()

    for q in range(KI):
        _fire_idx(q, q)
    for q in range(GA):
        _wait_fire_gather(q, q)

    def _visit(ci, _):
        g = lax.rem(ci, RK)
        islot = lax.rem(ci, KI)
        pltpu.make_async_copy(ya_hbm.at[sloc.at[0]], rows_v.at[g],
                              gsem.at[g]).wait()
        pltpu.sync_copy(rows_v.at[g], acc_sh.at[dloc.at[islot]], add=True)

        @pl.when(c == 1)
        def _():
            pltpu.sync_copy(ones_v, deg_sh.at[dloc.at[islot]], add=True)

        @pl.when(ci + KI < NCH)
        def _():
            _fire_idx(ci + KI, islot)

        @pl.when(ci + GA < NCH)
        def _():
            nxt = ci + GA
            _wait_fire_gather(lax.rem(nxt, KI), lax.rem(nxt, RK))
        return 0

    lax.fori_loop(0, NCH, _visit, 0)
    plsc.subcore_barrier()

    @pl.when(c == 0)
    def _():
        pltpu.sync_copy(acc_sh.at[pl.ds(s * rows_per_sub, rows_per_sub)],
                        acca_hbm.at[pl.ds(s * rows_per_sub, rows_per_sub)])

    @pl.when(c == 1)
    def _():
        pltpu.sync_copy(acc_sh.at[pl.ds(s * rows_per_sub, rows_per_sub)],
                        accb_hbm.at[pl.ds(s * rows_per_sub, rows_per_sub)])
        pltpu.sync_copy(deg_sh.at[pl.ds(s * deg_per_sub, deg_per_sub)],
                        deg_hbm.at[pl.ds(s * deg_per_sub, deg_per_sub)])


def kernel(x, edge_index, W0, b0, W1, b1):
    src = edge_index[0]
    dst = edge_index[1]
    y0a, y0b = _mm(x, W0)
    acc0a, acc0b, deg = _sc_agg(y0a, y0b, src, dst)
    deg_col = deg[:N, None]
    y1a, y1b = _layer_mm(acc0a, acc0b, deg_col, b0, W1)
    acc1a, acc1b, _ = _sc_agg(y1a, y1b, src, dst)
    return _final(acc1a, acc1b, deg_col, b1)
